# baseline (device time: 62260 ns/iter reference)
import functools

import jax
import jax.numpy as jnp
from jax import lax
from jax.experimental import pallas as pl
from jax.experimental.pallas import tpu as pltpu

N_DEV = 4
N_GLOBAL = 8192.0
EPS = 1e-5

M = 6144
BM = 768


def _stats_body(x_ref, out_ref):
    xb = x_ref[:, :]
    out_ref[:, 0:1] = jnp.sum(xb, axis=1, keepdims=True)
    out_ref[:, 1:2] = jnp.sum(xb * xb, axis=1, keepdims=True)


def _ring_body(stats_ref, out_ref, buf_ref, send_sems, recv_sems):
    my = lax.axis_index("i")
    left = lax.rem(my + (N_DEV - 1), N_DEV)
    right = lax.rem(my + 1, N_DEV)

    barrier_sem = pltpu.get_barrier_semaphore()
    for nbr in (left, right):
        pl.semaphore_signal(
            barrier_sem, inc=1, device_id=(nbr,),
            device_id_type=pl.DeviceIdType.MESH,
        )
    pl.semaphore_wait(barrier_sem, 2)

    buf_ref[0, :, :] = stats_ref[:, :]

    for h in range(N_DEV - 1):
        rdma = pltpu.make_async_remote_copy(
            src_ref=buf_ref.at[h],
            dst_ref=buf_ref.at[h + 1],
            send_sem=send_sems.at[h],
            recv_sem=recv_sems.at[h],
            device_id=(right,),
            device_id_type=pl.DeviceIdType.MESH,
        )
        rdma.start()
        rdma.wait()

    s = buf_ref[0] + buf_ref[1] + buf_ref[2] + buf_ref[3]
    mean = s[0:1, :] * (1.0 / N_GLOBAL)
    var = s[1:2, :] * (1.0 / N_GLOBAL) - mean * mean
    out_ref[0:1, :] = mean
    out_ref[1:2, :] = lax.rsqrt(var + EPS)

    @functools.partial(pl.run_scoped, sem=pltpu.SemaphoreType.REGULAR)
    def _(sem):
        for nbr in (left, right):
            pl.semaphore_signal(
                sem, inc=1, device_id=(nbr,),
                device_id_type=pl.DeviceIdType.MESH,
            )
        pl.semaphore_wait(sem, 2)


def _apply_body(x_ref, stats_ref, gamma_ref, beta_ref, out_ref):
    mean = stats_ref[:, 0:1]
    rstd = stats_ref[:, 1:2]
    g = gamma_ref[:, :]
    b = beta_ref[:, :]
    out_ref[:, :] = g * ((x_ref[:, :] - mean) * rstd) + b


def kernel(x, gamma, beta):
    m, n_loc = x.shape
    n_blocks = m // BM

    partial = pl.pallas_call(
        _stats_body,
        grid=(n_blocks,),
        in_specs=[pl.BlockSpec((BM, n_loc), lambda i: (i, 0))],
        out_specs=pl.BlockSpec((BM, 2), lambda i: (i, 0)),
        out_shape=jax.ShapeDtypeStruct((m, 2), jnp.float32),
    )(x)

    stats = pl.pallas_call(
        _ring_body,
        in_specs=[pl.BlockSpec(memory_space=pltpu.VMEM)],
        out_specs=pl.BlockSpec(memory_space=pltpu.VMEM),
        out_shape=jax.ShapeDtypeStruct((2, m), jnp.float32),
        scratch_shapes=[
            pltpu.VMEM((N_DEV, 2, m), jnp.float32),
            pltpu.SemaphoreType.DMA((N_DEV - 1,)),
            pltpu.SemaphoreType.DMA((N_DEV - 1,)),
        ],
        compiler_params=pltpu.CompilerParams(collective_id=0),
    )(partial.T)

    out = pl.pallas_call(
        _apply_body,
        grid=(n_blocks,),
        in_specs=[
            pl.BlockSpec((BM, n_loc), lambda i: (i, 0)),
            pl.BlockSpec((BM, 2), lambda i: (i, 0)),
            pl.BlockSpec((1, n_loc), lambda i: (0, 0)),
            pl.BlockSpec((1, n_loc), lambda i: (0, 0)),
        ],
        out_specs=pl.BlockSpec((BM, n_loc), lambda i: (i, 0)),
        out_shape=jax.ShapeDtypeStruct((m, n_loc), jnp.float32),
    )(x, stats.T, gamma.reshape(1, n_loc), beta.reshape(1, n_loc))

    return out
